# Initial kernel scaffold; baseline (speedup 1.0000x reference)
import jax
import jax.numpy as jnp
from jax.experimental import pallas as pl


def kernel(initializations, rel_emb, edge_index, edge_type):
    raise NotImplementedError("write your pallas kernel here")


# speedup vs baseline: 4.1280x; 4.1280x over previous
import functools

import jax
import jax.numpy as jnp
from jax import lax
from jax.experimental import pallas as pl
from jax.experimental.pallas import tpu as pltpu, tpu_sc as plsc

N_NODES = 10000
N_RELATIONS = 237
N_CHANNELS = 128
N_EDGES = 320000

NC = 2
NS = 16
NW = NC * NS
L = 16

B = 64
NB = 158
NPT = NB * B
E_PAD = NW * NPT
NJ = N_CHANNELS // L


def _tile_body(node_hbm, rel_hbm, nidx_hbm, ridx_hbm, out_hbm,
               nidx_v, ridx_v, rel_v, rows0_v, rows1_v,
               pbuf_v, scores_v, sem0, sem1):
    wid = lax.axis_index("s") * NC + lax.axis_index("c")

    pltpu.sync_copy(nidx_hbm.at[wid], nidx_v)
    pltpu.sync_copy(ridx_hbm.at[wid], ridx_v)
    pltpu.sync_copy(rel_hbm, rel_v)

    lane_iota = lax.iota(jnp.int32, L)

    def compute(b, rows_v):
        def group(g, _):
            base = g * L
            rv = ridx_v[b, pl.ds(base, L)]
            for e in range(L):
                row = base + e
                rid = rv[e]
                sl = pl.ds(0, L)
                acc = (rows_v[row, sl] * rel_v[rid, sl]) * rows_v[B + row, sl]
                for j in range(1, NJ):
                    sl = pl.ds(j * L, L)
                    acc = acc + (rows_v[row, sl] * rel_v[rid, sl]) * rows_v[B + row, sl]
                pbuf_v[e, :] = acc
            tot = plsc.load_gather(pbuf_v, [lane_iota, jnp.full((L,), 0, jnp.int32)])
            for i in range(1, L):
                col = jnp.full((L,), i, dtype=jnp.int32)
                tot = tot + plsc.load_gather(pbuf_v, [lane_iota, col])
            scores_v[b, pl.ds(base, L)] = tot
            return 0

        lax.fori_loop(0, B // L, group, 0, unroll=False)

    def fire(b, rows_v, sem):
        return pltpu.async_copy(node_hbm.at[nidx_v.at[b]], rows_v, sem)

    def wait(b, rows_v, sem):
        pltpu.make_async_copy(node_hbm.at[nidx_v.at[b]], rows_v, sem).wait()

    fire(0, rows0_v, sem0)

    def step(t, _):
        b0 = 2 * t
        b1 = b0 + 1
        fire(b1, rows1_v, sem1)
        wait(b0, rows0_v, sem0)
        compute(b0, rows0_v)

        @pl.when(b1 + 1 < NB)
        def _():
            fire(b1 + 1, rows0_v, sem0)

        wait(b1, rows1_v, sem1)
        compute(b1, rows1_v)
        return 0

    lax.fori_loop(0, NB // 2, step, 0, unroll=False)
    pltpu.sync_copy(scores_v, out_hbm.at[wid])


@jax.jit
def _dist_mult_sc(node_tbl, rel_tbl, nidx, ridx):
    mesh = plsc.VectorSubcoreMesh(core_axis_name="c", subcore_axis_name="s",
                                  num_cores=NC, num_subcores=NS)
    out = pl.kernel(
        _tile_body,
        out_type=jax.ShapeDtypeStruct((NW, NB, B), jnp.float32),
        mesh=mesh,
        compiler_params=pltpu.CompilerParams(needs_layout_passes=False),
        scratch_types=[
            pltpu.VMEM((NB, 2 * B), jnp.int32),
            pltpu.VMEM((NB, B), jnp.int32),
            pltpu.VMEM((N_RELATIONS, N_CHANNELS), jnp.float32),
            pltpu.VMEM((2 * B, N_CHANNELS), jnp.float32),
            pltpu.VMEM((2 * B, N_CHANNELS), jnp.float32),
            pltpu.VMEM((L, L), jnp.float32),
            pltpu.VMEM((NB, B), jnp.float32),
            pltpu.SemaphoreType.DMA,
            pltpu.SemaphoreType.DMA,
        ],
    )(node_tbl, rel_tbl, nidx, ridx)
    return out


def kernel(initializations, rel_emb, edge_index, edge_type):
    src = edge_index[0].astype(jnp.int32)
    dst = edge_index[1].astype(jnp.int32)
    rel = edge_type.astype(jnp.int32)
    pad = E_PAD - N_EDGES
    src = jnp.pad(src, (0, pad)).reshape(NW, NB, B)
    dst = jnp.pad(dst, (0, pad)).reshape(NW, NB, B)
    nidx = jnp.concatenate([src, dst], axis=-1)
    ridx = jnp.pad(rel, (0, pad)).reshape(NW, NB, B)
    out = _dist_mult_sc(initializations, rel_emb, nidx, ridx)
    return out.reshape(-1)[:N_EDGES]
